# baseline (speedup 1.0000x reference)
import functools

import jax
import jax.numpy as jnp
from jax import lax
from jax.experimental import pallas as pl
from jax.experimental.pallas import tpu as pltpu
from jax.experimental.pallas import tpu_sc as plsc

_LANES = 16
_MASK_PACK = 4


@functools.partial(jax.jit, static_argnums=(2, 3, 4, 5))
def _sc_lookup(tab_f, igbp, B, NCODES, D, L):
    info = plsc.get_sparse_core_info()
    NC, NS = info.num_cores, info.num_subcores
    NW = NC * NS
    bw = B // NW
    n_chunks = bw // _LANES

    mesh = plsc.VectorSubcoreMesh(core_axis_name="c", subcore_axis_name="s")

    @functools.partial(
        pl.kernel,
        mesh=mesh,
        compiler_params=pltpu.CompilerParams(
            use_tc_tiling_on_sc=False, needs_layout_passes=False),
        out_type=[
            jax.ShapeDtypeStruct((D, 1, B), jnp.float32),
            jax.ShapeDtypeStruct((L // 32, B // 128, 8, 128), jnp.int32),
        ],
        scratch_types=[
            pltpu.VMEM(((NCODES + 1) * D + _LANES,), jnp.float32),
            pltpu.VMEM((bw,), jnp.int32),
            pltpu.VMEM((D, 1, bw), jnp.float32),
            pltpu.VMEM((L // 32, bw // 128, 8, 128), jnp.int32),
        ],
    )
    def body(tab_h, igbp_h, emb_out, mask_out, tab_v, ig_v, ecols, mrep):
        wid = lax.axis_index("s") * NC + lax.axis_index("c")
        base = wid * bw

        pltpu.sync_copy(tab_h, tab_v.at[pl.ds(0, NCODES * D)])
        pltpu.sync_copy(igbp_h.at[pl.ds(base, bw)], ig_v)

        lanes = lax.iota(jnp.int32, _LANES)

        zeros = jnp.zeros((_LANES,), jnp.float32)
        for z in range((D + _LANES - 1) // _LANES):
            plsc.store_scatter(
                tab_v, [NCODES * D + z * _LANES + lanes], zeros)

        @plsc.parallel_loop(0, n_chunks, unroll=4)
        def _(i):
            off = i * _LANES
            ig = ig_v[pl.ds(off, _LANES)]
            valid = (ig >= 0) & (ig < NCODES)
            addr = jnp.where(valid, ig, NCODES) * D
            for d in range(D):
                col = plsc.load_gather(tab_v, [addr + d])
                ecols[d, 0, pl.ds(off, _LANES)] = col

        for ibl in range(bw // 128):
            for lg in range(128 // _LANES):
                ig = ig_v[pl.ds(ibl * 128 + lg * _LANES, _LANES)]
                nv = ((ig < 0) | (ig >= NCODES)).astype(jnp.int32)
                w = nv * 0x01010101
                for tl in range(L // 32):
                    for sg in range(8):
                        mrep[tl, ibl, sg, pl.ds(lg * _LANES, _LANES)] = w

        pltpu.sync_copy(ecols, emb_out.at[:, :, pl.ds(base, bw)])
        pltpu.sync_copy(
            mrep, mask_out.at[:, pl.ds(wid * (bw // 128), bw // 128), :, :])

    return body(tab_f, igbp)


def kernel(igbp, predictor_values, emb_table):
    B = igbp.shape[0]
    L = predictor_values.shape[1]
    num_codes, D = emb_table.shape
    emb_t, mask_w = _sc_lookup(
        emb_table.reshape(-1), igbp, B, num_codes, D, L)
    emb = emb_t.transpose(2, 1, 0)
    mask = (
        mask_w.view(jnp.uint8)
        .reshape(L // 32, B // 128, 8, 128, 4)
        .transpose(1, 3, 0, 2, 4)
        .reshape(B, L, 1)
        .view(jnp.bool_)
    )
    return emb, mask


# speedup vs baseline: 1.0029x; 1.0029x over previous
import functools

import jax
import jax.numpy as jnp
from jax import lax
from jax.experimental import pallas as pl
from jax.experimental.pallas import tpu as pltpu
from jax.experimental.pallas import tpu_sc as plsc

_LANES = 16
_MASK_PACK = 4


@functools.partial(jax.jit, static_argnums=(2, 3, 4, 5))
def _sc_lookup(tab_f, igbp, B, NCODES, D, L):
    info = plsc.get_sparse_core_info()
    NC, NS = info.num_cores, info.num_subcores
    NW = NC * NS
    bw = B // NW
    n_chunks = bw // _LANES

    mesh = plsc.VectorSubcoreMesh(core_axis_name="c", subcore_axis_name="s")

    @functools.partial(
        pl.kernel,
        mesh=mesh,
        compiler_params=pltpu.CompilerParams(
            use_tc_tiling_on_sc=False, needs_layout_passes=False),
        out_type=[
            jax.ShapeDtypeStruct((D, 1, B), jnp.float32),
            jax.ShapeDtypeStruct((L // 32, B // 128, 8, 128), jnp.int32),
        ],
        scratch_types=[
            pltpu.VMEM(((NCODES + 1) * D + _LANES,), jnp.float32),
            pltpu.VMEM((bw,), jnp.int32),
            pltpu.VMEM((D, 1, bw), jnp.float32),
            pltpu.VMEM((L // 32, bw // 128, 8, 128), jnp.int32),
        ],
    )
    def body(tab_h, igbp_h, emb_out, mask_out, tab_v, ig_v, ecols, mrep):
        wid = lax.axis_index("s") * NC + lax.axis_index("c")
        base = wid * bw

        pltpu.sync_copy(tab_h, tab_v.at[pl.ds(0, NCODES * D)])
        pltpu.sync_copy(igbp_h.at[pl.ds(base, bw)], ig_v)

        lanes = lax.iota(jnp.int32, _LANES)

        zeros = jnp.zeros((_LANES,), jnp.float32)
        for z in range((D + _LANES - 1) // _LANES):
            plsc.store_scatter(
                tab_v, [NCODES * D + z * _LANES + lanes], zeros)

        @plsc.parallel_loop(0, n_chunks, unroll=2)
        def _(i):
            off = i * _LANES
            ig = ig_v[pl.ds(off, _LANES)]
            valid = (ig >= 0) & (ig < NCODES)
            addr = jnp.where(valid, ig, NCODES) * D
            for d in range(D):
                col = plsc.load_gather(tab_v, [addr + d])
                ecols[d, 0, pl.ds(off, _LANES)] = col

        for ibl in range(bw // 128):
            for lg in range(128 // _LANES):
                ig = ig_v[pl.ds(ibl * 128 + lg * _LANES, _LANES)]
                nv = ((ig < 0) | (ig >= NCODES)).astype(jnp.int32)
                w = nv * 0x01010101
                for tl in range(L // 32):
                    for sg in range(8):
                        mrep[tl, ibl, sg, pl.ds(lg * _LANES, _LANES)] = w

        pltpu.sync_copy(ecols, emb_out.at[:, :, pl.ds(base, bw)])
        pltpu.sync_copy(
            mrep, mask_out.at[:, pl.ds(wid * (bw // 128), bw // 128), :, :])

    return body(tab_f, igbp)


def kernel(igbp, predictor_values, emb_table):
    B = igbp.shape[0]
    L = predictor_values.shape[1]
    num_codes, D = emb_table.shape
    emb_t, mask_w = _sc_lookup(
        emb_table.reshape(-1), igbp, B, num_codes, D, L)
    emb = emb_t.transpose(2, 1, 0)
    mask = (
        mask_w.view(jnp.uint8)
        .reshape(L // 32, B // 128, 8, 128, 4)
        .transpose(1, 3, 0, 2, 4)
        .reshape(B, L, 1)
        .view(jnp.bool_)
    )
    return emb, mask


# speedup vs baseline: 1.0448x; 1.0417x over previous
import functools

import jax
import jax.numpy as jnp
from jax import lax
from jax.experimental import pallas as pl
from jax.experimental.pallas import tpu as pltpu
from jax.experimental.pallas import tpu_sc as plsc

_LANES = 16
_MASK_PACK = 4


@functools.partial(jax.jit, static_argnums=(2, 3, 4, 5))
def _sc_lookup(tab_f, igbp, B, NCODES, D, L):
    info = plsc.get_sparse_core_info()
    NC, NS = info.num_cores, info.num_subcores
    NW = NC * NS
    bw = B // NW
    n_chunks = bw // _LANES

    mesh = plsc.VectorSubcoreMesh(core_axis_name="c", subcore_axis_name="s")

    @functools.partial(
        pl.kernel,
        mesh=mesh,
        compiler_params=pltpu.CompilerParams(
            use_tc_tiling_on_sc=False, needs_layout_passes=False),
        out_type=[
            jax.ShapeDtypeStruct((D, 1, B), jnp.float32),
            jax.ShapeDtypeStruct((L // 32, B // 128, 8, 128), jnp.int32),
        ],
        scratch_types=[
            pltpu.VMEM(((NCODES + 1) * D + _LANES,), jnp.float32),
            pltpu.VMEM((bw,), jnp.int32),
            pltpu.VMEM((D, 1, bw), jnp.float32),
            pltpu.VMEM((L // 32, bw // 128, 8, 128), jnp.int32),
        ],
    )
    def body(tab_h, igbp_h, emb_out, mask_out, tab_v, ig_v, ecols, mrep):
        wid = lax.axis_index("s") * NC + lax.axis_index("c")
        base = wid * bw

        pltpu.sync_copy(tab_h, tab_v.at[pl.ds(0, NCODES * D)])
        pltpu.sync_copy(igbp_h.at[pl.ds(base, bw)], ig_v)

        lanes = lax.iota(jnp.int32, _LANES)

        zeros = jnp.zeros((_LANES,), jnp.float32)
        for z in range((D + _LANES - 1) // _LANES):
            plsc.store_scatter(
                tab_v, [NCODES * D + z * _LANES + lanes], zeros)

        @plsc.parallel_loop(0, n_chunks, unroll=2)
        def _(i):
            off = i * _LANES
            ig = ig_v[pl.ds(off, _LANES)]
            valid = (ig >= 0) & (ig < NCODES)
            addr = jnp.where(valid, ig, NCODES) * D
            for d in range(D):
                col = plsc.load_gather(tab_v, [addr + d])
                ecols[d, 0, pl.ds(off, _LANES)] = col

        for ibl in range(bw // 128):
            for lg in range(128 // _LANES):
                ig = ig_v[pl.ds(ibl * 128 + lg * _LANES, _LANES)]
                nv = ((ig < 0) | (ig >= NCODES)).astype(jnp.int32)
                w = nv * 0x01010101
                for tl in range(L // 32):
                    for sg in range(8):
                        mrep[tl, ibl, sg, pl.ds(lg * _LANES, _LANES)] = w

        pltpu.sync_copy(ecols, emb_out.at[:, :, pl.ds(base, bw)])
        pltpu.sync_copy(
            mrep, mask_out.at[:, pl.ds(wid * (bw // 128), bw // 128), :, :])

    return body(tab_f, igbp)


def kernel(igbp, predictor_values, emb_table):
    B = igbp.shape[0]
    L = predictor_values.shape[1]
    num_codes, D = emb_table.shape
    emb_t, mask_w = _sc_lookup(
        emb_table.reshape(-1), igbp, B, num_codes, D, L)
    emb = emb_t.transpose(2, 1, 0)
    mask = jnp.broadcast_to(
        (mask_w != 0).transpose(1, 3, 0, 2)[:, :, :, :, None],
        (B // 128, 128, L // 32, 8, _MASK_PACK),
    ).reshape(B, L, 1)
    return emb, mask


# speedup vs baseline: 1.3872x; 1.3277x over previous
import functools

import jax
import jax.numpy as jnp
from jax import lax
from jax.experimental import pallas as pl
from jax.experimental.pallas import tpu as pltpu
from jax.experimental.pallas import tpu_sc as plsc

_LANES = 16


@functools.partial(jax.jit, static_argnums=(2, 3, 4))
def _sc_gather(tab_f, igbp, B, NCODES, D):
    info = plsc.get_sparse_core_info()
    NC, NS = info.num_cores, info.num_subcores
    NW = NC * NS
    bw = B // NW
    n_chunks = bw // _LANES

    mesh = plsc.VectorSubcoreMesh(core_axis_name="c", subcore_axis_name="s")

    @functools.partial(
        pl.kernel,
        mesh=mesh,
        compiler_params=pltpu.CompilerParams(
            use_tc_tiling_on_sc=False, needs_layout_passes=False),
        out_type=jax.ShapeDtypeStruct((D, 1, B), jnp.float32),
        scratch_types=[
            pltpu.VMEM(((NCODES + 1) * D + _LANES,), jnp.float32),
            pltpu.VMEM((bw,), jnp.int32),
            pltpu.VMEM((D, 1, bw), jnp.float32),
        ],
    )
    def body(tab_h, igbp_h, emb_out, tab_v, ig_v, ecols):
        wid = lax.axis_index("s") * NC + lax.axis_index("c")
        base = wid * bw

        pltpu.sync_copy(tab_h, tab_v.at[pl.ds(0, NCODES * D)])
        pltpu.sync_copy(igbp_h.at[pl.ds(base, bw)], ig_v)

        lanes = lax.iota(jnp.int32, _LANES)

        zeros = jnp.zeros((_LANES,), jnp.float32)
        for z in range((D + _LANES - 1) // _LANES):
            plsc.store_scatter(
                tab_v, [NCODES * D + z * _LANES + lanes], zeros)

        @plsc.parallel_loop(0, n_chunks, unroll=2)
        def _(i):
            off = i * _LANES
            ig = ig_v[pl.ds(off, _LANES)]
            valid = (ig >= 0) & (ig < NCODES)
            addr = jnp.where(valid, ig, NCODES) * D
            for d in range(D):
                col = plsc.load_gather(tab_v, [addr + d])
                ecols[d, 0, pl.ds(off, _LANES)] = col

        pltpu.sync_copy(ecols, emb_out.at[:, :, pl.ds(base, bw)])

    return body(tab_f, igbp)


@functools.partial(jax.jit, static_argnums=(1, 2, 3, 4))
def _tc_mask(igbp2, B, NCODES, L, R):
    def body(ig_ref, out_ref):
        for r in range(R):
            ig = ig_ref[pl.ds(r, 1), :]
            nv = ((ig < 0) | (ig >= NCODES)).astype(jnp.uint8)
            out_ref[:, pl.ds(r * (B // R), B // R)] = jnp.broadcast_to(
                nv, (L, B // R))

    return pl.pallas_call(
        body,
        out_shape=jax.ShapeDtypeStruct((L, B), jnp.uint8),
    )(igbp2)


def kernel(igbp, predictor_values, emb_table):
    B = igbp.shape[0]
    L = predictor_values.shape[1]
    num_codes, D = emb_table.shape
    R = 8
    emb_t = _sc_gather(emb_table.reshape(-1), igbp, B, num_codes, D)
    mask_t = _tc_mask(igbp.reshape(R, B // R), B, num_codes, L, R)
    emb = emb_t.transpose(2, 1, 0)
    mask = mask_t.T[:, :, None].view(jnp.bool_)
    return emb, mask
